# Initial kernel scaffold; baseline (speedup 1.0000x reference)
import jax
import jax.numpy as jnp
from jax.experimental import pallas as pl


def kernel(x, edge_index, edge_attr, Wq1, bq1, Wk1, bk1, Wv1, bv1, We1, Ws1, bs1, Wq2, bq2, Wk2, bk2, Wv2, bv2, We2, Ws2, bs2, Wfc, bfc):
    raise NotImplementedError("write your pallas kernel here")


# speedup vs baseline: 11.7651x; 11.7651x over previous
import functools

import jax
import jax.numpy as jnp
from jax import lax
from jax.experimental import pallas as pl
from jax.experimental.pallas import tpu as pltpu
from jax.experimental.pallas import tpu_sc as plsc

N = 100000
E = 3200000
D = 16
NP = 100352
NC = 2
NS = 16
NW = NC * NS
EPT = E // NW
BG = 2000
NCHUNK = EPT // BG
BS = 800
RPT = NP // NS
ZROWS = RPT // 8
BN = 1024
GRID = NP // BN
BE = 6400
EGRID = E // BE

_SC_PARAMS = pltpu.CompilerParams(use_tc_tiling_on_sc=False)
_SC_MESH = plsc.VectorSubcoreMesh(
    core_axis_name="c", subcore_axis_name="s", num_cores=NC, num_subcores=NS)


def _gather_body(q4_hbm, k_hbm, v_hbm, src_hbm, dst_hbm,
                 qd_out, ks_out, vs_out,
                 src_v, dst_v, qb, kb, vb):
    cid = lax.axis_index("c")
    sid = lax.axis_index("s")
    ebase = (cid * NS + sid) * EPT

    def _chunk(c, carry):
        coff = ebase + c * BG
        pltpu.sync_copy(src_hbm.at[pl.ds(coff, BG)], src_v)
        pltpu.sync_copy(dst_hbm.at[pl.ds(coff, BG)], dst_v)
        pltpu.sync_copy(q4_hbm.at[dst_v], qb)
        pltpu.sync_copy(k_hbm.at[src_v], kb)
        pltpu.sync_copy(v_hbm.at[src_v], vb)
        pltpu.sync_copy(qb, qd_out.at[pl.ds(coff, BG)])
        pltpu.sync_copy(kb, ks_out.at[pl.ds(coff, BG)])
        pltpu.sync_copy(vb, vs_out.at[pl.ds(coff, BG)])
        return carry
    lax.fori_loop(0, NCHUNK, _chunk, 0)


_gather_call = functools.partial(
    pl.kernel,
    out_type=[jax.ShapeDtypeStruct((E, D), jnp.float32)] * 3,
    mesh=_SC_MESH,
    scratch_types=[
        pltpu.VMEM((BG,), jnp.int32),
        pltpu.VMEM((BG,), jnp.int32),
        pltpu.VMEM((BG, D), jnp.float32),
        pltpu.VMEM((BG, D), jnp.float32),
        pltpu.VMEM((BG, D), jnp.float32),
    ],
    compiler_params=_SC_PARAMS,
)(_gather_body)


def _scatter_body(dst_hbm, wv_hbm, w_hbm,
                  a_out, sw_out,
                  dst_v, wvb, wb, a_sp, sw_sp):
    cid = lax.axis_index("c")
    sid = lax.axis_index("s")
    z16 = jnp.zeros((16,), jnp.float32)

    def _z2(i, c):
        wvb[i, :] = z16
        return c
    lax.fori_loop(0, ZROWS, _z2, 0)

    def _z1(i, c):
        wb[pl.ds(i * 16, 16)] = z16
        return c
    lax.fori_loop(0, ZROWS // 16, _z1, 0)

    nbase = sid * RPT
    for j in range(8):
        pltpu.sync_copy(wvb.at[pl.ds(0, ZROWS)],
                        a_sp.at[pl.ds(nbase + j * ZROWS, ZROWS)])
        pltpu.sync_copy(wb.at[pl.ds(0, ZROWS)],
                        sw_sp.at[pl.ds(nbase + j * ZROWS, ZROWS)])
    plsc.subcore_barrier()

    ebase = (cid * NS + sid) * EPT

    def _chunk(c, carry):
        coff = ebase + c * BS
        pltpu.sync_copy(dst_hbm.at[pl.ds(coff, BS)], dst_v)
        pltpu.sync_copy(wv_hbm.at[pl.ds(coff, BS)], wvb)
        pltpu.sync_copy(w_hbm.at[pl.ds(coff, BS)], wb)
        pltpu.sync_copy(wvb, a_sp.at[dst_v], add=True)
        pltpu.sync_copy(wb, sw_sp.at[dst_v], add=True)
        return carry
    lax.fori_loop(0, EPT // BS, _chunk, 0)

    plsc.subcore_barrier()

    obase = cid * NP + sid * RPT
    for j in range(8):
        pltpu.sync_copy(a_sp.at[pl.ds(nbase + j * ZROWS, ZROWS)],
                        wvb.at[pl.ds(0, ZROWS)])
        pltpu.sync_copy(wvb.at[pl.ds(0, ZROWS)],
                        a_out.at[pl.ds(obase + j * ZROWS, ZROWS)])
        pltpu.sync_copy(sw_sp.at[pl.ds(nbase + j * ZROWS, ZROWS)],
                        wb.at[pl.ds(0, ZROWS)])
        pltpu.sync_copy(wb.at[pl.ds(0, ZROWS)],
                        sw_out.at[pl.ds(obase + j * ZROWS, ZROWS)])


_scatter_call = functools.partial(
    pl.kernel,
    out_type=[
        jax.ShapeDtypeStruct((NC * NP, D), jnp.float32),
        jax.ShapeDtypeStruct((NC * NP,), jnp.float32),
    ],
    mesh=_SC_MESH,
    scratch_types=[
        pltpu.VMEM((BS,), jnp.int32),
        pltpu.VMEM((BS, D), jnp.float32),
        pltpu.VMEM((BS,), jnp.float32),
        pltpu.VMEM_SHARED((NP, D), jnp.float32),
        pltpu.VMEM_SHARED((NP,), jnp.float32),
    ],
    compiler_params=_SC_PARAMS,
)(_scatter_body)


def _proj(h, w_ref, b_ref):
    return jnp.dot(h, w_ref[...], preferred_element_type=jnp.float32) + b_ref[...]


def _kmax_update(i, kmax_o, k):
    bmax = jnp.max(jnp.abs(k)).reshape(1, 1)

    @pl.when(i == 0)
    def _():
        kmax_o[...] = bmax

    @pl.when(i > 0)
    def _():
        kmax_o[...] = jnp.maximum(kmax_o[...], bmax)


def _amax_body(ea_ref, amax_o):
    _kmax_update(pl.program_id(0), amax_o, ea_ref[...])


def _prologue_body(x_ref, wq, bq, wk, bk, wv, bv, ws, bs,
                   q4_o, k_o, v_o, s_o, kmax_o):
    x = x_ref[...]
    q4_o[...] = _proj(x, wq, bq) * 0.25
    k = _proj(x, wk, bk)
    k_o[...] = k
    v_o[...] = _proj(x, wv, bv)
    s_o[...] = _proj(x, ws, bs)
    _kmax_update(pl.program_id(0), kmax_o, k)


def _edge_body(qd_ref, ks_ref, vs_ref, ea_ref, par,
               wv_o, w_o):
    qd = qd_ref[...]
    ks = ks_ref[...]
    ea = ea_ref[...]
    we_b = par[0:1, :]
    bnd_b = par[1:2, :]
    logit = (jnp.sum(qd * ks, axis=1, keepdims=True)
             + ea * jnp.sum(qd * we_b, axis=1, keepdims=True))
    bound = jnp.sum(jnp.abs(qd) * bnd_b, axis=1, keepdims=True)
    w = jnp.exp(logit - bound)
    wv_o[...] = (vs_ref[...] + ea * we_b) * w
    w_o[...] = w


def _merge_h(a0, a1, sw0, sw1, skip):
    a = a0[...] + a1[...]
    sw = sw0[...] + sw1[...] + 1e-16
    return jnp.maximum(a / sw + skip[...], 0.0)


def _mid_body(a0, a1, sw0, sw1, skip,
              wq, bq, wk, bk, wv, bv, ws, bs,
              q4_o, k_o, v_o, s_o, kmax_o):
    h = _merge_h(a0, a1, sw0, sw1, skip)
    q4_o[...] = _proj(h, wq, bq) * 0.25
    k = _proj(h, wk, bk)
    k_o[...] = k
    v_o[...] = _proj(h, wv, bv)
    s_o[...] = _proj(h, ws, bs)
    _kmax_update(pl.program_id(0), kmax_o, k)


def _head_body(a0, a1, sw0, sw1, skip,
               x_ref, wfc, bfc, o_ref):
    h = _merge_h(a0, a1, sw0, sw1, skip)
    o = _proj(h, wfc, bfc)
    nrm = jnp.sqrt(jnp.sum(o * o, axis=1, keepdims=True))
    o = o / jnp.maximum(nrm, 1e-12) * 10.0
    x = x_ref[...]
    colid = lax.broadcasted_iota(jnp.int32, (BN, D), 1)
    lm = (x[:, 3:4] == -1.0) & (colid == 0)
    um = (x[:, 5:6] == 1.0) & (colid == 2)
    o = o + jnp.where(lm, -10.0, 0.0) + jnp.where(um, -10.0, 0.0)
    o_ref[...] = o


def _rspec(r, d):
    return pl.BlockSpec((r, d), lambda i: (i, 0))


def _full_spec(shape):
    return pl.BlockSpec(shape, lambda i: tuple(0 for _ in shape))


def _node_out_types():
    return [jax.ShapeDtypeStruct((NP, D), jnp.float32)] * 4 + [
        jax.ShapeDtypeStruct((1, 1), jnp.float32)]


def _node_out_specs():
    return [_rspec(BN, D)] * 4 + [_full_spec((1, 1))]


_W16 = _full_spec((D, D))
_B16 = _full_spec((1, D))

_prologue_call = pl.pallas_call(
    _prologue_body,
    grid=(GRID,),
    in_specs=[_rspec(BN, 6)] + [_full_spec((6, D)), _B16] * 4,
    out_specs=_node_out_specs(),
    out_shape=_node_out_types(),
)

_amax_call = pl.pallas_call(
    _amax_body,
    grid=(EGRID,),
    in_specs=[_rspec(BE, 1)],
    out_specs=_full_spec((1, 1)),
    out_shape=jax.ShapeDtypeStruct((1, 1), jnp.float32),
)

_edge_call = pl.pallas_call(
    _edge_body,
    grid=(EGRID,),
    in_specs=[_rspec(BE, D)] * 3 + [_rspec(BE, 1), _full_spec((2, D))],
    out_specs=[_rspec(BE, D), _rspec(BE, 1)],
    out_shape=[jax.ShapeDtypeStruct((E, D), jnp.float32),
               jax.ShapeDtypeStruct((E, 1), jnp.float32)],
)

_merge_specs = [_rspec(BN, D), _rspec(BN, D), _rspec(BN, 1), _rspec(BN, 1),
                _rspec(BN, D)]

_mid_call = pl.pallas_call(
    _mid_body,
    grid=(GRID,),
    in_specs=_merge_specs + [_W16, _B16] * 4,
    out_specs=_node_out_specs(),
    out_shape=_node_out_types(),
)

_head_call = pl.pallas_call(
    _head_body,
    grid=(GRID,),
    in_specs=_merge_specs + [_rspec(BN, 6), _W16, _B16],
    out_specs=_rspec(BN, D),
    out_shape=jax.ShapeDtypeStruct((NP, D), jnp.float32),
)


def _params(we_row, kmax, amax):
    bnd = kmax.reshape(1)[0] + amax.reshape(1)[0] * jnp.abs(we_row)
    return jnp.stack([we_row, bnd])


def _split(arr2d, arr_sw):
    return (arr2d[:NP], arr2d[NP:],
            arr_sw[:NP].reshape(NP, 1), arr_sw[NP:].reshape(NP, 1))


def _layer(q4, k, v, src, dst, ea2d, par):
    qd, ks, vs = _gather_call(q4, k, v, src, dst)
    wv, w = _edge_call(qd, ks, vs, ea2d, par)
    return _scatter_call(dst, wv, w.reshape(E))


def kernel(x, edge_index, edge_attr, Wq1, bq1, Wk1, bk1, Wv1, bv1, We1, Ws1, bs1,
           Wq2, bq2, Wk2, bk2, Wv2, bv2, We2, Ws2, bs2, Wfc, bfc):
    f32 = jnp.float32
    xp = jnp.concatenate([x, jnp.zeros((NP - N, 6), f32)], axis=0)
    src = edge_index[0]
    dst = edge_index[1]

    amax = _amax_call(edge_attr)
    r1 = lambda b: b.reshape(1, D)
    q41, k1, v1, s1, kmax1 = _prologue_call(
        xp, Wq1, r1(bq1), Wk1, r1(bk1), Wv1, r1(bv1), Ws1, r1(bs1))
    a1, sw1 = _layer(q41, k1, v1, src, dst, edge_attr,
                     _params(We1[0], kmax1, amax))

    q42, k2, v2, s2, kmax2 = _mid_call(
        *_split(a1, sw1), s1,
        Wq2, r1(bq2), Wk2, r1(bk2), Wv2, r1(bv2), Ws2, r1(bs2))
    a2, sw2 = _layer(q42, k2, v2, src, dst, edge_attr,
                     _params(We2[0], kmax2, amax))

    wfc_p = jnp.concatenate([Wfc, jnp.zeros((D, D - 3), f32)], axis=1)
    bfc_p = jnp.concatenate([bfc, jnp.zeros((D - 3,), f32)]).reshape(1, D)
    o = _head_call(*_split(a2, sw2), s2, xp, wfc_p, bfc_p)
    return o[:N - 1, :3]


# speedup vs baseline: 43.3978x; 3.6887x over previous
import functools

import jax
import jax.numpy as jnp
from jax import lax
from jax.experimental import pallas as pl
from jax.experimental.pallas import tpu as pltpu
from jax.experimental.pallas import tpu_sc as plsc

N = 100000
E = 3200000
D = 16
NP = 100352
NC = 2
NS = 16
NW = NC * NS
EPT = E // NW
BG = 2000
NCHUNK = EPT // BG
BS = 1000
E8 = E // 8
RPT = NP // NS
ZROWS = RPT // 8
BN = 1024
GRID = NP // BN
BE = 6400
EGRID = E // BE

_SC_PARAMS = pltpu.CompilerParams(use_tc_tiling_on_sc=False)
_SC_MESH = plsc.VectorSubcoreMesh(
    core_axis_name="c", subcore_axis_name="s", num_cores=NC, num_subcores=NS)


def _gather_body(q4_hbm, k_hbm, v_hbm, src_hbm, dst_hbm,
                 qd_out, ks_out, vs_out,
                 src_v, dst_v, qb, kb, vb):
    cid = lax.axis_index("c")
    sid = lax.axis_index("s")
    ebase = (cid * NS + sid) * EPT

    def _chunk(c, carry):
        coff = ebase + c * BG
        pltpu.sync_copy(src_hbm.at[pl.ds(coff, BG)], src_v)
        pltpu.sync_copy(dst_hbm.at[pl.ds(coff, BG)], dst_v)
        pltpu.sync_copy(q4_hbm.at[dst_v], qb)
        pltpu.sync_copy(k_hbm.at[src_v], kb)
        pltpu.sync_copy(v_hbm.at[src_v], vb)
        pltpu.sync_copy(qb, qd_out.at[pl.ds(coff, BG)])
        pltpu.sync_copy(kb, ks_out.at[pl.ds(coff, BG)])
        pltpu.sync_copy(vb, vs_out.at[pl.ds(coff, BG)])
        return carry
    lax.fori_loop(0, NCHUNK, _chunk, 0)


_gather_call = functools.partial(
    pl.kernel,
    out_type=[jax.ShapeDtypeStruct((E, D), jnp.float32)] * 3,
    mesh=_SC_MESH,
    scratch_types=[
        pltpu.VMEM((BG,), jnp.int32),
        pltpu.VMEM((BG,), jnp.int32),
        pltpu.VMEM((BG, D), jnp.float32),
        pltpu.VMEM((BG, D), jnp.float32),
        pltpu.VMEM((BG, D), jnp.float32),
    ],
    compiler_params=_SC_PARAMS,
)(_gather_body)


def _scatter_body(dst_hbm, rows_hbm, a_out, dst_v, rb, a_sp):
    cid = lax.axis_index("c")
    sid = lax.axis_index("s")
    z16 = jnp.zeros((16,), jnp.float32)

    def _z2(i, c):
        rb[i, :] = z16
        return c
    lax.fori_loop(0, ZROWS, _z2, 0)

    nbase = sid * RPT
    for j in range(8):
        pltpu.sync_copy(rb.at[pl.ds(0, ZROWS)],
                        a_sp.at[pl.ds(nbase + j * ZROWS, ZROWS)])
    plsc.subcore_barrier()

    ebase = (cid * NS + sid) * EPT

    def _chunk(c, carry):
        coff = ebase + c * BS
        pltpu.sync_copy(dst_hbm.at[pl.ds(coff, BS)], dst_v)
        pltpu.sync_copy(rows_hbm.at[pl.ds(coff, BS)], rb)
        pltpu.sync_copy(rb, a_sp.at[dst_v], add=True)
        return carry
    lax.fori_loop(0, EPT // BS, _chunk, 0)

    plsc.subcore_barrier()

    obase = cid * NP + sid * RPT
    for j in range(8):
        pltpu.sync_copy(a_sp.at[pl.ds(nbase + j * ZROWS, ZROWS)],
                        rb.at[pl.ds(0, ZROWS)])
        pltpu.sync_copy(rb.at[pl.ds(0, ZROWS)],
                        a_out.at[pl.ds(obase + j * ZROWS, ZROWS)])


_scatter_call = functools.partial(
    pl.kernel,
    out_type=jax.ShapeDtypeStruct((NC * NP, D), jnp.float32),
    mesh=_SC_MESH,
    scratch_types=[
        pltpu.VMEM((BS,), jnp.int32),
        pltpu.VMEM((BS, D), jnp.float32),
        pltpu.VMEM_SHARED((NP, D), jnp.float32),
    ],
    compiler_params=_SC_PARAMS,
)(_scatter_body)


def _proj(h, w_ref, b_ref):
    return jnp.dot(h, w_ref[...], preferred_element_type=jnp.float32) + b_ref[...]


def _kmax_update(i, kmax_o, k):
    bmax = jnp.max(jnp.abs(k)).reshape(1, 1)

    @pl.when(i == 0)
    def _():
        kmax_o[...] = bmax

    @pl.when(i > 0)
    def _():
        kmax_o[...] = jnp.maximum(kmax_o[...], bmax)


def _amax_body(ea_ref, amax_o):
    _kmax_update(pl.program_id(0), amax_o, ea_ref[...])


AB = 1000


def _prologue_body(x_ref, wq, bq, wk, bk, wv, bv, ws, bs,
                   q4_o, k_o, v_o, s_o, kmax_o):
    x = x_ref[...]
    q4_o[...] = _proj(x, wq, bq) * 0.25
    k = _proj(x, wk, bk)
    k_o[...] = k
    v_o[...] = _proj(x, wv, bv)
    s_o[...] = _proj(x, ws, bs)
    _kmax_update(pl.program_id(0), kmax_o, k)


def _edge_body(qd_ref, ks_ref, vs_ref, eab_ref, par,
               wv_o, w_o):
    ra = lax.broadcasted_iota(jnp.int32, (128, 128), 0) // D
    cb = lax.broadcasted_iota(jnp.int32, (128, 128), 1) // D
    S = jnp.where(ra == cb, 1.0, 0.0).astype(jnp.float32)
    qd = qd_ref[...]
    ks = ks_ref[...]
    vs = vs_ref[...]
    eab = eab_ref[...]
    we_t = par[0:1, :]
    bnd_t = par[1:2, :]

    def gsum(x):
        return jnp.dot(x, S, preferred_element_type=jnp.float32)

    logit = gsum(qd * ks) + eab * gsum(qd * we_t)
    w = jnp.exp(logit - gsum(jnp.abs(qd) * bnd_t))
    wv_o[...] = (vs + eab * we_t) * w
    w_o[...] = w


def _merge_h(a0, a1, sw0, sw1, skip):
    a = a0[...] + a1[...]
    sw = sw0[:, :1] + sw1[:, :1] + 1e-16
    return jnp.maximum(a / sw + skip[...], 0.0)


def _mid_body(a0, a1, sw0, sw1, skip,
              wq, bq, wk, bk, wv, bv, ws, bs,
              q4_o, k_o, v_o, s_o, kmax_o):
    h = _merge_h(a0, a1, sw0, sw1, skip)
    q4_o[...] = _proj(h, wq, bq) * 0.25
    k = _proj(h, wk, bk)
    k_o[...] = k
    v_o[...] = _proj(h, wv, bv)
    s_o[...] = _proj(h, ws, bs)
    _kmax_update(pl.program_id(0), kmax_o, k)


def _head_body(a0, a1, sw0, sw1, skip,
               x_ref, wfc, bfc, o_ref):
    h = _merge_h(a0, a1, sw0, sw1, skip)
    o = _proj(h, wfc, bfc)
    nrm = jnp.sqrt(jnp.sum(o * o, axis=1, keepdims=True))
    o = o / jnp.maximum(nrm, 1e-12) * 10.0
    x = x_ref[...]
    colid = lax.broadcasted_iota(jnp.int32, (BN, D), 1)
    lm = (x[:, 3:4] == -1.0) & (colid == 0)
    um = (x[:, 5:6] == 1.0) & (colid == 2)
    o = o + jnp.where(lm, -10.0, 0.0) + jnp.where(um, -10.0, 0.0)
    o_ref[...] = o


def _rspec(r, d):
    return pl.BlockSpec((r, d), lambda i: (i, 0))


def _full_spec(shape):
    return pl.BlockSpec(shape, lambda i: tuple(0 for _ in shape))


def _node_out_types():
    return [jax.ShapeDtypeStruct((NP, D), jnp.float32)] * 4 + [
        jax.ShapeDtypeStruct((1, 1), jnp.float32)]


def _node_out_specs():
    return [_rspec(BN, D)] * 4 + [_full_spec((1, 1))]


_W16 = _full_spec((D, D))
_B16 = _full_spec((1, D))

_prologue_call = pl.pallas_call(
    _prologue_body,
    grid=(GRID,),
    in_specs=[_rspec(BN, 6)] + [_full_spec((6, D)), _B16] * 4,
    out_specs=_node_out_specs(),
    out_shape=_node_out_types(),
)

_amax_call = pl.pallas_call(
    _amax_body,
    grid=((E // 128) // AB,),
    in_specs=[_rspec(AB, 128)],
    out_specs=_full_spec((1, 1)),
    out_shape=jax.ShapeDtypeStruct((1, 1), jnp.float32),
)

EB8 = BE // 8
_edge_call = pl.pallas_call(
    _edge_body,
    grid=(EGRID,),
    in_specs=[_rspec(EB8, 128)] * 4 + [_full_spec((2, 128))],
    out_specs=[_rspec(EB8, 128)] * 2,
    out_shape=[jax.ShapeDtypeStruct((E8, 128), jnp.float32)] * 2,
)

_merge_specs = [_rspec(BN, D)] * 5

_mid_call = pl.pallas_call(
    _mid_body,
    grid=(GRID,),
    in_specs=_merge_specs + [_W16, _B16] * 4,
    out_specs=_node_out_specs(),
    out_shape=_node_out_types(),
)

_head_call = pl.pallas_call(
    _head_body,
    grid=(GRID,),
    in_specs=_merge_specs + [_rspec(BN, 6), _W16, _B16],
    out_specs=_rspec(BN, D),
    out_shape=jax.ShapeDtypeStruct((NP, D), jnp.float32),
)


def _params(we_row, kmax, amax):
    bnd = kmax.reshape(1)[0] + amax.reshape(1)[0] * jnp.abs(we_row)
    return jnp.stack([jnp.tile(we_row, 8), jnp.tile(bnd, 8)])


def _split(arr2d, arr_sw):
    return (arr2d[:NP], arr2d[NP:], arr_sw[:NP], arr_sw[NP:])


def _layer(q4, k, v, src, dst, eabp, par):
    qd, ks, vs = _gather_call(q4, k, v, src, dst)
    p8 = lambda t: t.reshape(E8, 128)
    wv, w16 = _edge_call(p8(qd), p8(ks), p8(vs), eabp, par)
    a = _scatter_call(dst, wv.reshape(E, D))
    sw = _scatter_call(dst, w16.reshape(E, D))
    return a, sw


def kernel(x, edge_index, edge_attr, Wq1, bq1, Wk1, bk1, Wv1, bv1, We1, Ws1, bs1,
           Wq2, bq2, Wk2, bk2, Wv2, bv2, We2, Ws2, bs2, Wfc, bfc):
    f32 = jnp.float32
    xp = jnp.concatenate([x, jnp.zeros((NP - N, 6), f32)], axis=0)
    src = edge_index[0]
    dst = edge_index[1]

    ea_flat = edge_attr.reshape(E)
    amax = _amax_call(ea_flat.reshape(E // 128, 128))
    eabp = jnp.broadcast_to(ea_flat[:, None], (E, D)).reshape(E8, 128)
    r1 = lambda b: b.reshape(1, D)
    q41, k1, v1, s1, kmax1 = _prologue_call(
        xp, Wq1, r1(bq1), Wk1, r1(bk1), Wv1, r1(bv1), Ws1, r1(bs1))
    a1, sw1 = _layer(q41, k1, v1, src, dst, eabp,
                     _params(We1[0], kmax1, amax))

    q42, k2, v2, s2, kmax2 = _mid_call(
        *_split(a1, sw1), s1,
        Wq2, r1(bq2), Wk2, r1(bk2), Wv2, r1(bv2), Ws2, r1(bs2))
    a2, sw2 = _layer(q42, k2, v2, src, dst, eabp,
                     _params(We2[0], kmax2, amax))

    wfc_p = jnp.concatenate([Wfc, jnp.zeros((D, D - 3), f32)], axis=1)
    bfc_p = jnp.concatenate([bfc, jnp.zeros((D - 3,), f32)]).reshape(1, D)
    o = _head_call(*_split(a2, sw2), s2, xp, wfc_p, bfc_p)
    return o[:N - 1, :3]


# speedup vs baseline: 48.0269x; 1.1067x over previous
import functools

import jax
import jax.numpy as jnp
from jax import lax
from jax.experimental import pallas as pl
from jax.experimental.pallas import tpu as pltpu
from jax.experimental.pallas import tpu_sc as plsc

N = 100000
E = 3200000
D = 16
NP = 100352
NC = 2
NS = 16
NW = NC * NS
EPT = E // NW
BG = 1000
NCHUNK = EPT // BG
BS = 400
E8 = E // 8
RPT = NP // NS
ZROWS = RPT // 8
BN = 1024
GRID = NP // BN
BE = 6400
EGRID = E // BE

_SC_PARAMS = pltpu.CompilerParams(use_tc_tiling_on_sc=False)
_SC_MESH = plsc.VectorSubcoreMesh(
    core_axis_name="c", subcore_axis_name="s", num_cores=NC, num_subcores=NS)


def _gather_body(q4_hbm, k_hbm, v_hbm, src_hbm, dst_hbm,
                 qd_out, ks_out, vs_out,
                 src0, dst0, qb0, kb0, vb0, src1, dst1, qb1, kb1, vb1,
                 gs0, gs1, ws0, ws1):
    cid = lax.axis_index("c")
    sid = lax.axis_index("s")
    ebase = (cid * NS + sid) * EPT
    bufs = ((src0, dst0, qb0, kb0, vb0, gs0, ws0),
            (src1, dst1, qb1, kb1, vb1, gs1, ws1))

    def _start(c, p):
        srcv, dstv, qb, kb, vb, gs, _ = bufs[p]
        coff = ebase + c * BG
        pltpu.sync_copy(src_hbm.at[pl.ds(coff, BG)], srcv)
        pltpu.sync_copy(dst_hbm.at[pl.ds(coff, BG)], dstv)
        pltpu.async_copy(q4_hbm.at[dstv], qb, gs)
        pltpu.async_copy(k_hbm.at[srcv], kb, gs)
        pltpu.async_copy(v_hbm.at[srcv], vb, gs)

    def _flush(c, p):
        srcv, dstv, qb, kb, vb, gs, ws = bufs[p]
        coff = ebase + c * BG
        pltpu.make_async_copy(q4_hbm.at[dstv], qb, gs).wait()
        pltpu.make_async_copy(k_hbm.at[srcv], kb, gs).wait()
        pltpu.make_async_copy(v_hbm.at[srcv], vb, gs).wait()
        pltpu.async_copy(qb, qd_out.at[pl.ds(coff, BG)], ws)
        pltpu.async_copy(kb, ks_out.at[pl.ds(coff, BG)], ws)
        pltpu.async_copy(vb, vs_out.at[pl.ds(coff, BG)], ws)

    def _drainw(p):
        srcv, dstv, qb, kb, vb, gs, ws = bufs[p]
        pltpu.make_async_copy(qb, qd_out.at[pl.ds(0, BG)], ws).wait()
        pltpu.make_async_copy(kb, ks_out.at[pl.ds(0, BG)], ws).wait()
        pltpu.make_async_copy(vb, vs_out.at[pl.ds(0, BG)], ws).wait()

    _start(0, 0)
    _start(1, 1)

    def _pair(i, carry):
        c0 = i * 2

        @pl.when(i > 0)
        def _():
            _flush(c0 - 2, 0)
            _flush(c0 - 1, 1)
            _drainw(0)
            _drainw(1)
            _start(c0, 0)
            _start(c0 + 1, 1)
        return carry
    lax.fori_loop(1, NCHUNK // 2, _pair, 0)
    _flush(NCHUNK - 2, 0)
    _flush(NCHUNK - 1, 1)
    _drainw(0)
    _drainw(1)


_gather_call = functools.partial(
    pl.kernel,
    out_type=[jax.ShapeDtypeStruct((E, D), jnp.float32)] * 3,
    mesh=_SC_MESH,
    scratch_types=[
        pltpu.VMEM((BG,), jnp.int32),
        pltpu.VMEM((BG,), jnp.int32),
        pltpu.VMEM((BG, D), jnp.float32),
        pltpu.VMEM((BG, D), jnp.float32),
        pltpu.VMEM((BG, D), jnp.float32),
        pltpu.VMEM((BG,), jnp.int32),
        pltpu.VMEM((BG,), jnp.int32),
        pltpu.VMEM((BG, D), jnp.float32),
        pltpu.VMEM((BG, D), jnp.float32),
        pltpu.VMEM((BG, D), jnp.float32),
        pltpu.SemaphoreType.DMA,
        pltpu.SemaphoreType.DMA,
        pltpu.SemaphoreType.DMA,
        pltpu.SemaphoreType.DMA,
    ],
    compiler_params=_SC_PARAMS,
)(_gather_body)


def _scatter_body(dst_hbm, rows_hbm, a_out,
                  dst0, rb0, dst1, rb1, ls0, ls1, ss0, ss1, a_sp):
    cid = lax.axis_index("c")
    sid = lax.axis_index("s")
    z16 = jnp.zeros((16,), jnp.float32)

    def _z2(i, c):
        rb0[i, :] = z16
        return c
    lax.fori_loop(0, ZROWS, _z2, 0)

    nbase = sid * RPT
    for j in range(8):
        pltpu.sync_copy(rb0.at[pl.ds(0, ZROWS)],
                        a_sp.at[pl.ds(nbase + j * ZROWS, ZROWS)])
    plsc.subcore_barrier()

    ebase = (cid * NS + sid) * EPT
    bufs = ((dst0, rb0, ls0, ss0), (dst1, rb1, ls1, ss1))

    def _load(c, p):
        dstv, rb, ls, _ = bufs[p]
        coff = ebase + c * BS
        pltpu.async_copy(dst_hbm.at[pl.ds(coff, BS)], dstv, ls)
        pltpu.async_copy(rows_hbm.at[pl.ds(coff, BS)], rb, ls)

    def _scat(c, p):
        dstv, rb, ls, ss = bufs[p]
        coff = ebase + c * BS
        pltpu.make_async_copy(dst_hbm.at[pl.ds(coff, BS)], dstv, ls).wait()
        pltpu.make_async_copy(rows_hbm.at[pl.ds(coff, BS)], rb, ls).wait()
        pltpu.async_copy(rb, a_sp.at[dstv], ss, add=True)

    def _drain(p):
        dstv, rb, ls, ss = bufs[p]
        pltpu.make_async_copy(rb, a_sp.at[dstv], ss).wait()

    _load(0, 0)
    _load(1, 1)

    def _pair2(i, carry):
        c0 = i * 2

        @pl.when(i > 0)
        def _():
            _drain(0)
            _drain(1)
            _load(c0, 0)
            _load(c0 + 1, 1)
        _scat(c0, 0)
        _scat(c0 + 1, 1)
        return carry
    lax.fori_loop(0, (EPT // BS) // 2, _pair2, 0)
    _drain(0)
    _drain(1)

    plsc.subcore_barrier()

    obase = cid * NP + sid * RPT
    for j in range(8):
        pltpu.sync_copy(a_sp.at[pl.ds(nbase + j * ZROWS, ZROWS)],
                        rb0.at[pl.ds(0, ZROWS)])
        pltpu.sync_copy(rb0.at[pl.ds(0, ZROWS)],
                        a_out.at[pl.ds(obase + j * ZROWS, ZROWS)])


_scatter_call = functools.partial(
    pl.kernel,
    out_type=jax.ShapeDtypeStruct((NC * NP, D), jnp.float32),
    mesh=_SC_MESH,
    scratch_types=[
        pltpu.VMEM((BS,), jnp.int32),
        pltpu.VMEM((BS, D), jnp.float32),
        pltpu.VMEM((BS,), jnp.int32),
        pltpu.VMEM((BS, D), jnp.float32),
        pltpu.SemaphoreType.DMA,
        pltpu.SemaphoreType.DMA,
        pltpu.SemaphoreType.DMA,
        pltpu.SemaphoreType.DMA,
        pltpu.VMEM_SHARED((NP, D), jnp.float32),
    ],
    compiler_params=_SC_PARAMS,
)(_scatter_body)


def _proj(h, w_ref, b_ref):
    return jnp.dot(h, w_ref[...], preferred_element_type=jnp.float32) + b_ref[...]


def _kmax_update(i, kmax_o, k):
    bmax = jnp.max(jnp.abs(k)).reshape(1, 1)

    @pl.when(i == 0)
    def _():
        kmax_o[...] = bmax

    @pl.when(i > 0)
    def _():
        kmax_o[...] = jnp.maximum(kmax_o[...], bmax)


def _amax_body(ea_ref, amax_o):
    _kmax_update(pl.program_id(0), amax_o, ea_ref[...])


AB = 1000


def _prologue_body(x_ref, wq, bq, wk, bk, wv, bv, ws, bs,
                   q4_o, k_o, v_o, s_o, kmax_o):
    x = x_ref[...]
    q4_o[...] = _proj(x, wq, bq) * 0.25
    k = _proj(x, wk, bk)
    k_o[...] = k
    v_o[...] = _proj(x, wv, bv)
    s_o[...] = _proj(x, ws, bs)
    _kmax_update(pl.program_id(0), kmax_o, k)


def _edge_body(qd_ref, ks_ref, vs_ref, eab_ref, par,
               wv_o, w_o):
    ra = lax.broadcasted_iota(jnp.int32, (128, 128), 0) // D
    cb = lax.broadcasted_iota(jnp.int32, (128, 128), 1) // D
    S = jnp.where(ra == cb, 1.0, 0.0).astype(jnp.float32)
    qd = qd_ref[...]
    ks = ks_ref[...]
    vs = vs_ref[...]
    eab = eab_ref[...]
    we_t = par[0:1, :]
    bnd_t = par[1:2, :]

    def gsum(x):
        return jnp.dot(x, S, preferred_element_type=jnp.float32)

    logit = gsum(qd * ks) + eab * gsum(qd * we_t)
    w = jnp.exp(logit - gsum(jnp.abs(qd) * bnd_t))
    wv_o[...] = (vs + eab * we_t) * w
    w_o[...] = w


def _merge_h(a0, a1, sw0, sw1, skip):
    a = a0[...] + a1[...]
    sw = sw0[:, :1] + sw1[:, :1] + 1e-16
    return jnp.maximum(a / sw + skip[...], 0.0)


def _mid_body(a0, a1, sw0, sw1, skip,
              wq, bq, wk, bk, wv, bv, ws, bs,
              q4_o, k_o, v_o, s_o, kmax_o):
    h = _merge_h(a0, a1, sw0, sw1, skip)
    q4_o[...] = _proj(h, wq, bq) * 0.25
    k = _proj(h, wk, bk)
    k_o[...] = k
    v_o[...] = _proj(h, wv, bv)
    s_o[...] = _proj(h, ws, bs)
    _kmax_update(pl.program_id(0), kmax_o, k)


def _head_body(a0, a1, sw0, sw1, skip,
               x_ref, wfc, bfc, o_ref):
    h = _merge_h(a0, a1, sw0, sw1, skip)
    o = _proj(h, wfc, bfc)
    nrm = jnp.sqrt(jnp.sum(o * o, axis=1, keepdims=True))
    o = o / jnp.maximum(nrm, 1e-12) * 10.0
    x = x_ref[...]
    colid = lax.broadcasted_iota(jnp.int32, (BN, D), 1)
    lm = (x[:, 3:4] == -1.0) & (colid == 0)
    um = (x[:, 5:6] == 1.0) & (colid == 2)
    o = o + jnp.where(lm, -10.0, 0.0) + jnp.where(um, -10.0, 0.0)
    o_ref[...] = o


def _rspec(r, d):
    return pl.BlockSpec((r, d), lambda i: (i, 0))


def _full_spec(shape):
    return pl.BlockSpec(shape, lambda i: tuple(0 for _ in shape))


def _node_out_types():
    return [jax.ShapeDtypeStruct((NP, D), jnp.float32)] * 4 + [
        jax.ShapeDtypeStruct((1, 1), jnp.float32)]


def _node_out_specs():
    return [_rspec(BN, D)] * 4 + [_full_spec((1, 1))]


_W16 = _full_spec((D, D))
_B16 = _full_spec((1, D))

_prologue_call = pl.pallas_call(
    _prologue_body,
    grid=(GRID,),
    in_specs=[_rspec(BN, 6)] + [_full_spec((6, D)), _B16] * 4,
    out_specs=_node_out_specs(),
    out_shape=_node_out_types(),
)

_amax_call = pl.pallas_call(
    _amax_body,
    grid=((E // 128) // AB,),
    in_specs=[_rspec(AB, 128)],
    out_specs=_full_spec((1, 1)),
    out_shape=jax.ShapeDtypeStruct((1, 1), jnp.float32),
)

EB8 = BE // 8
_edge_call = pl.pallas_call(
    _edge_body,
    grid=(EGRID,),
    in_specs=[_rspec(EB8, 128)] * 4 + [_full_spec((2, 128))],
    out_specs=[_rspec(EB8, 128)] * 2,
    out_shape=[jax.ShapeDtypeStruct((E8, 128), jnp.float32)] * 2,
)

_merge_specs = [_rspec(BN, D)] * 5

_mid_call = pl.pallas_call(
    _mid_body,
    grid=(GRID,),
    in_specs=_merge_specs + [_W16, _B16] * 4,
    out_specs=_node_out_specs(),
    out_shape=_node_out_types(),
)

_head_call = pl.pallas_call(
    _head_body,
    grid=(GRID,),
    in_specs=_merge_specs + [_rspec(BN, 6), _W16, _B16],
    out_specs=_rspec(BN, D),
    out_shape=jax.ShapeDtypeStruct((NP, D), jnp.float32),
)


def _params(we_row, kmax, amax):
    bnd = kmax.reshape(1)[0] + amax.reshape(1)[0] * jnp.abs(we_row)
    return jnp.stack([jnp.tile(we_row, 8), jnp.tile(bnd, 8)])


def _split(arr2d, arr_sw):
    return (arr2d[:NP], arr2d[NP:], arr_sw[:NP], arr_sw[NP:])


def _layer(q4, k, v, src, dst, eabp, par):
    qd, ks, vs = _gather_call(q4, k, v, src, dst)
    p8 = lambda t: t.reshape(E8, 128)
    wv, w16 = _edge_call(p8(qd), p8(ks), p8(vs), eabp, par)
    a = _scatter_call(dst, wv.reshape(E, D))
    sw = _scatter_call(dst, w16.reshape(E, D))
    return a, sw


def kernel(x, edge_index, edge_attr, Wq1, bq1, Wk1, bk1, Wv1, bv1, We1, Ws1, bs1,
           Wq2, bq2, Wk2, bk2, Wv2, bv2, We2, Ws2, bs2, Wfc, bfc):
    f32 = jnp.float32
    xp = jnp.concatenate([x, jnp.zeros((NP - N, 6), f32)], axis=0)
    src = edge_index[0]
    dst = edge_index[1]

    ea_flat = edge_attr.reshape(E)
    amax = _amax_call(ea_flat.reshape(E // 128, 128))
    eabp = jnp.broadcast_to(ea_flat[:, None], (E, D)).reshape(E8, 128)
    r1 = lambda b: b.reshape(1, D)
    q41, k1, v1, s1, kmax1 = _prologue_call(
        xp, Wq1, r1(bq1), Wk1, r1(bk1), Wv1, r1(bv1), Ws1, r1(bs1))
    a1, sw1 = _layer(q41, k1, v1, src, dst, eabp,
                     _params(We1[0], kmax1, amax))

    q42, k2, v2, s2, kmax2 = _mid_call(
        *_split(a1, sw1), s1,
        Wq2, r1(bq2), Wk2, r1(bk2), Wv2, r1(bv2), Ws2, r1(bs2))
    a2, sw2 = _layer(q42, k2, v2, src, dst, eabp,
                     _params(We2[0], kmax2, amax))

    wfc_p = jnp.concatenate([Wfc, jnp.zeros((D, D - 3), f32)], axis=1)
    bfc_p = jnp.concatenate([bfc, jnp.zeros((D - 3,), f32)]).reshape(1, D)
    o = _head_call(*_split(a2, sw2), s2, xp, wfc_p, bfc_p)
    return o[:N - 1, :3]
